# baseline (speedup 1.0000x reference)
import jax
import jax.numpy as jnp
from jax.experimental import pallas as pl

TAU = 0.07
TOPK = 10


def _exp_kernel(x_ref, o_ref):
    o_ref[...] = jnp.exp(x_ref[...] / TAU)


def kernel(h, peaks, labels):
    h_n = h / jnp.linalg.norm(h, axis=-1, keepdims=True)
    p_n = peaks / jnp.linalg.norm(peaks, axis=-1, keepdims=True)
    sim = jnp.matmul(h_n, p_n.T)
    exp_dot = pl.pallas_call(
        _exp_kernel,
        grid=(16,),
        in_specs=[pl.BlockSpec((256, 8192), lambda i: (i, 0))],
        out_specs=pl.BlockSpec((256, 8192), lambda i: (i, 0)),
        out_shape=jax.ShapeDtypeStruct(sim.shape, sim.dtype),
    )(sim)
    numerators = jnp.matmul(exp_dot, labels)
    denominators = jnp.sum(exp_dot, axis=-1)
    p_i = numerators / denominators[:, None]
    top_sims, knn_indices = jax.lax.top_k(sim, TOPK)
    density = jnp.mean(top_sims, axis=-1)
    return p_i, density, knn_indices


# speedup vs baseline: 3.0002x; 3.0002x over previous
import functools

import jax
import jax.numpy as jnp
from jax.experimental import pallas as pl
from jax.experimental.pallas import tpu as pltpu

TAU = 0.07
TOPK = 10


def _snn_kernel(hb_ref, ptb_ref, lb_ref, pi_ref, aux_ref, knn_ref,
                acc_ref, simT_ref, *, n_p, qb, pb, c_real, p_total):
    p = pl.program_id(1)

    sim = jnp.dot(hb_ref[...], ptb_ref[...],
                  preferred_element_type=jnp.float32)
    simT_ref[pl.ds(p * pb, pb), :] = sim.T

    e = jnp.exp(sim * (1.0 / TAU))
    contrib = jnp.dot(e.astype(jnp.bfloat16), lb_ref[...],
                      preferred_element_type=jnp.float32)

    @pl.when(p == 0)
    def _init():
        acc_ref[...] = contrib

    @pl.when(p > 0)
    def _accum():
        acc_ref[...] += contrib

    @pl.when(p == n_p - 1)
    def _finalize():
        acc = acc_ref[...]
        cpad = acc.shape[1]
        iota_c = jax.lax.broadcasted_iota(jnp.int32, (qb, cpad), 1)
        den = jnp.sum(jnp.where(iota_c == c_real, acc, 0.0), axis=1,
                      keepdims=True)
        pi_ref[...] = acc / den

        iota_p = jax.lax.broadcasted_iota(jnp.int32, (p_total, qb), 0)
        dens = jnp.zeros((1, qb), jnp.float32)
        idx_rows = []
        for k in range(TOPK):
            x = simT_ref[...]
            m = jnp.max(x, axis=0, keepdims=True)
            cand = jnp.where(x == m, iota_p, p_total)
            i = jnp.min(cand, axis=0, keepdims=True)
            dens = dens + m
            idx_rows.append(i)
            if k < TOPK - 1:
                simT_ref[...] = jnp.where(iota_p == i, -jnp.inf, x)
        aux_ref[...] = jnp.broadcast_to(dens / float(TOPK), aux_ref.shape)
        pad = 16 - TOPK
        idx_rows.append(jnp.zeros((pad, qb), jnp.int32))
        knn_ref[...] = jnp.concatenate(idx_rows, axis=0)


def kernel(h, peaks, labels):
    q, d = h.shape
    p_total, c = labels.shape[0], labels.shape[1]

    h_n = h / jnp.linalg.norm(h, axis=-1, keepdims=True)
    p_n = peaks / jnp.linalg.norm(peaks, axis=-1, keepdims=True)

    hb = h_n.astype(jnp.bfloat16)
    ptb = p_n.astype(jnp.bfloat16).T

    cpad = ((c + 1 + 127) // 128) * 128
    lb = jnp.pad(labels.astype(jnp.bfloat16), ((0, 0), (0, cpad - c)))
    ones_col = (jax.lax.broadcasted_iota(jnp.int32, (1, cpad), 1) == c)
    lb = jnp.where(ones_col, jnp.bfloat16(1.0), lb)

    qb = 512 if q % 512 == 0 else q
    pb = 1024 if p_total % 1024 == 0 else p_total
    n_q, n_p = q // qb, p_total // pb

    body = functools.partial(_snn_kernel, n_p=n_p, qb=qb, pb=pb,
                             c_real=c, p_total=p_total)
    pi_pad, aux, knn_t = pl.pallas_call(
        body,
        grid=(n_q, n_p),
        in_specs=[
            pl.BlockSpec((qb, d), lambda iq, ip: (iq, 0)),
            pl.BlockSpec((d, pb), lambda iq, ip: (0, ip)),
            pl.BlockSpec((pb, cpad), lambda iq, ip: (ip, 0)),
        ],
        out_specs=[
            pl.BlockSpec((qb, cpad), lambda iq, ip: (iq, 0)),
            pl.BlockSpec((8, qb), lambda iq, ip: (0, iq)),
            pl.BlockSpec((16, qb), lambda iq, ip: (0, iq)),
        ],
        out_shape=[
            jax.ShapeDtypeStruct((q, cpad), jnp.float32),
            jax.ShapeDtypeStruct((8, q), jnp.float32),
            jax.ShapeDtypeStruct((16, q), jnp.int32),
        ],
        scratch_shapes=[
            pltpu.VMEM((qb, cpad), jnp.float32),
            pltpu.VMEM((p_total, qb), jnp.float32),
        ],
    )(hb, ptb, lb)

    p_i = pi_pad[:, :c]
    density = aux[0]
    knn_indices = knn_t[:TOPK].T
    return p_i, density, knn_indices


# speedup vs baseline: 3.9235x; 1.3077x over previous
import functools

import jax
import jax.numpy as jnp
from jax.experimental import pallas as pl
from jax.experimental.pallas import tpu as pltpu

TAU = 0.07
TOPK = 10


def _topk_iteration(k, simT_ref, buf, iota_ref, aux_ref, knn_ref, qb, p_total):
    x = simT_ref[buf]
    iota_p = iota_ref[...]
    m = jnp.max(x, axis=0, keepdims=True)
    cand = jnp.where(x == m, iota_p, p_total)
    i = jnp.min(cand, axis=0, keepdims=True)
    if k == 0:
        aux_ref[0:1, :] = m
    else:
        aux_ref[0:1, :] += m
    knn_ref[k:k + 1, :] = i
    if k < TOPK - 1:
        simT_ref[buf] = jnp.where(iota_p == i, -jnp.inf, x)
    else:
        aux_ref[0:1, :] = aux_ref[0:1, :] / float(TOPK)


def _snn_kernel(hb_ref, ptb_ref, lb_ref, pi_ref, aux_ref, knn_ref,
                acc_ref, simT_ref, iota_ref, *, n_q, n_p, qb, pb, c_real,
                p_total):
    iq = pl.program_id(0)
    ip = pl.program_id(1)

    @pl.when(jnp.logical_and(iq == 0, ip == 0))
    def _init_iota():
        iota_ref[...] = jax.lax.broadcasted_iota(jnp.int32, (p_total, qb), 0)

    @pl.when(iq < n_q)
    def _compute():
        sim = jnp.dot(hb_ref[...], ptb_ref[...],
                      preferred_element_type=jnp.float32)
        simT_ref[iq % 2, pl.ds(ip * pb, pb), :] = sim.T

        e = jnp.exp(sim * (1.0 / TAU))
        contrib = jnp.dot(e.astype(jnp.bfloat16), lb_ref[...],
                          preferred_element_type=jnp.float32)

        @pl.when(ip == 0)
        def _init():
            acc_ref[...] = contrib

        @pl.when(ip > 0)
        def _accum():
            acc_ref[...] += contrib

        @pl.when(ip == n_p - 1)
        def _finalize_pi():
            acc = acc_ref[...]
            cpad = acc.shape[1]
            iota_c = jax.lax.broadcasted_iota(jnp.int32, (qb, cpad), 1)
            den = jnp.sum(jnp.where(iota_c == c_real, acc, 0.0), axis=1,
                          keepdims=True)
            pi_ref[...] = acc / den

    per_step = -(-TOPK // n_p)
    prev_buf = (iq + 1) % 2

    @pl.when(iq > 0)
    def _topk():
        for c in range((TOPK + per_step - 1) // per_step):
            ks = [k for k in range(c * per_step, min((c + 1) * per_step, TOPK))]

            @pl.when(ip == c)
            def _run(ks=ks):
                for k in ks:
                    _topk_iteration(k, simT_ref, prev_buf, iota_ref,
                                    aux_ref, knn_ref, qb, p_total)


def kernel(h, peaks, labels):
    q, d = h.shape
    p_total, c = labels.shape[0], labels.shape[1]

    h_n = h / jnp.linalg.norm(h, axis=-1, keepdims=True)
    p_n = peaks / jnp.linalg.norm(peaks, axis=-1, keepdims=True)

    hb = h_n.astype(jnp.bfloat16)
    ptb = p_n.astype(jnp.bfloat16).T

    cpad = ((c + 1 + 127) // 128) * 128
    lb = jnp.pad(labels.astype(jnp.bfloat16), ((0, 0), (0, cpad - c)))
    ones_col = (jax.lax.broadcasted_iota(jnp.int32, (1, cpad), 1) == c)
    lb = jnp.where(ones_col, jnp.bfloat16(1.0), lb)

    qb = 256 if q % 256 == 0 else q
    pb = 1024 if p_total % 1024 == 0 else p_total
    n_q, n_p = q // qb, p_total // pb

    body = functools.partial(_snn_kernel, n_q=n_q, n_p=n_p, qb=qb, pb=pb,
                             c_real=c, p_total=p_total)
    last_q = n_q - 1
    pi_pad, aux, knn_t = pl.pallas_call(
        body,
        grid=(n_q + 1, n_p),
        in_specs=[
            pl.BlockSpec((qb, d), lambda iq, ip: (jnp.minimum(iq, last_q), 0)),
            pl.BlockSpec((d, pb), lambda iq, ip: (0, ip)),
            pl.BlockSpec((pb, cpad), lambda iq, ip: (ip, 0)),
        ],
        out_specs=[
            pl.BlockSpec((qb, cpad),
                         lambda iq, ip: (jnp.minimum(iq, last_q), 0)),
            pl.BlockSpec((8, qb), lambda iq, ip: (0, jnp.maximum(iq - 1, 0))),
            pl.BlockSpec((16, qb), lambda iq, ip: (0, jnp.maximum(iq - 1, 0))),
        ],
        out_shape=[
            jax.ShapeDtypeStruct((q, cpad), jnp.float32),
            jax.ShapeDtypeStruct((8, q), jnp.float32),
            jax.ShapeDtypeStruct((16, q), jnp.int32),
        ],
        scratch_shapes=[
            pltpu.VMEM((qb, cpad), jnp.float32),
            pltpu.VMEM((2, p_total, qb), jnp.float32),
            pltpu.VMEM((p_total, qb), jnp.int32),
        ],
    )(hb, ptb, lb)

    p_i = pi_pad[:, :c]
    density = aux[0]
    knn_indices = knn_t[:TOPK].T
    return p_i, density, knn_indices
